# baseline (speedup 1.0000x reference)
import functools

import jax
import jax.numpy as jnp
from jax.experimental import pallas as pl
from jax.experimental.pallas import tpu as pltpu


def _unary_kernel(xt_ref, out_ref, *, L, Cb):
    c0 = pl.program_id(0) * Cb
    xt = xt_ref[...]
    Bb = xt.shape[1]
    ci = jax.lax.broadcasted_iota(jnp.int32, (Cb, L, Bb), 0) + c0
    ii = jax.lax.broadcasted_iota(jnp.int32, (Cb, L, Bb), 1)
    out_ref[...] = ((ci == ii) | (ci - L == xt[None, :, :])).astype(jnp.float32)


def _binary_kernel(xi_ref, xt_ref, out_ref, *, L, Ib):
    i0 = pl.program_id(0) * Ib
    xi = xi_ref[...]
    xt = xt_ref[...]
    U, V = xt.shape[1], xt.shape[2]
    shape = (Ib, L - 1, U, V)
    kk = jax.lax.broadcasted_iota(jnp.int32, shape, 1)
    ii = jax.lax.broadcasted_iota(jnp.int32, shape, 0) + i0
    xk0 = xt[None, : L - 1, :, :]
    xk1 = xt[None, 1:L, :, :]
    xj = jnp.where(kk < ii, xk0, xk1)
    out_ref[...] = (xi[:, None, :, :] == xj).astype(jnp.float32)


@jax.jit
def _run(inputs):
    B, L = inputs.shape
    S1 = 101
    C = L + S1
    xt = inputs.T
    params = pltpu.CompilerParams(dimension_semantics=("arbitrary",))

    Cb = 4
    unary_t = pl.pallas_call(
        functools.partial(_unary_kernel, L=L, Cb=Cb),
        grid=(pl.cdiv(C, Cb),),
        in_specs=[pl.BlockSpec((L, B), lambda c: (0, 0))],
        out_specs=pl.BlockSpec((Cb, L, B), lambda c: (c, 0, 0)),
        out_shape=jax.ShapeDtypeStruct((C, L, B), jnp.float32),
        compiler_params=params,
    )(xt)

    Ib = 4
    U, V = B // 128, 128
    xt4 = xt.reshape(L, U, V)
    binary_t = pl.pallas_call(
        functools.partial(_binary_kernel, L=L, Ib=Ib),
        grid=(pl.cdiv(L, Ib),),
        in_specs=[
            pl.BlockSpec((Ib, U, V), lambda i: (i, 0, 0)),
            pl.BlockSpec((L, U, V), lambda i: (0, 0, 0)),
        ],
        out_specs=pl.BlockSpec((Ib, L - 1, U, V), lambda i: (i, 0, 0, 0)),
        out_shape=jax.ShapeDtypeStruct((L, L - 1, U, V), jnp.float32),
        compiler_params=params,
    )(xt4, xt4)

    unary = jnp.transpose(unary_t, (2, 1, 0))
    binary = (
        jnp.transpose(binary_t, (2, 3, 0, 1)).reshape(B, L, L - 1)[..., None]
    )
    return unary, binary


def kernel(inputs, table):
    del table
    return _run(inputs)


# speedup vs baseline: 1.0996x; 1.0996x over previous
import functools

import jax
import jax.numpy as jnp
from jax.experimental import pallas as pl
from jax.experimental.pallas import tpu as pltpu


def _fused(xt_ref, xt4_ref, xi_ref, unary_ref, binary_ref, *, L, Cb, Ib, NB):
    j = pl.program_id(0)

    @pl.when(j < NB)
    def _binary():
        i0 = j * Ib
        xt = xt4_ref[...]
        U, V = xt.shape[1], xt.shape[2]
        shape = (Ib, L - 1, U, V)
        kk = jax.lax.broadcasted_iota(jnp.int32, shape, 1)
        ii = jax.lax.broadcasted_iota(jnp.int32, shape, 0) + i0
        xk0 = xt[None, : L - 1, :, :]
        xk1 = xt[None, 1:L, :, :]
        xj = jnp.where(kk < ii, xk0, xk1)
        xi = xi_ref[...]
        binary_ref[...] = (xi[:, None, :, :] == xj).astype(jnp.float32)

    @pl.when(j >= NB)
    def _unary():
        c0 = (j - NB) * Cb
        xt = xt_ref[...]
        Bb = xt.shape[1]
        ci = jax.lax.broadcasted_iota(jnp.int32, (Cb, L, Bb), 0) + c0
        ii = jax.lax.broadcasted_iota(jnp.int32, (Cb, L, Bb), 1)
        unary_ref[...] = ((ci == ii) | (ci - L == xt[None, :, :])).astype(
            jnp.float32
        )


@jax.jit
def _run(inputs):
    B, L = inputs.shape
    S1 = 101
    C = L + S1
    xt = inputs.T
    U, V = B // 128, 128
    xt4 = xt.reshape(L, U, V)
    Cb, Ib = 8, 8
    NB = pl.cdiv(L, Ib)
    NU = pl.cdiv(C, Cb)
    unary_t, binary_t = pl.pallas_call(
        functools.partial(_fused, L=L, Cb=Cb, Ib=Ib, NB=NB),
        grid=(NB + NU,),
        in_specs=[
            pl.BlockSpec((L, B), lambda j: (0, 0)),
            pl.BlockSpec((L, U, V), lambda j: (0, 0, 0)),
            pl.BlockSpec((Ib, U, V), lambda j: (jnp.minimum(j, NB - 1), 0, 0)),
        ],
        out_specs=[
            pl.BlockSpec((Cb, L, B), lambda j: (jnp.maximum(j - NB, 0), 0, 0)),
            pl.BlockSpec(
                (Ib, L - 1, U, V), lambda j: (jnp.minimum(j, NB - 1), 0, 0, 0)
            ),
        ],
        out_shape=[
            jax.ShapeDtypeStruct((C, L, B), jnp.float32),
            jax.ShapeDtypeStruct((L, L - 1, U, V), jnp.float32),
        ],
        compiler_params=pltpu.CompilerParams(
            dimension_semantics=("arbitrary",)
        ),
    )(xt, xt4, xt4)
    unary = jnp.transpose(unary_t, (2, 1, 0))
    binary = (
        jnp.transpose(binary_t, (2, 3, 0, 1)).reshape(B, L, L - 1)[..., None]
    )
    return unary, binary


def kernel(inputs, table):
    del table
    return _run(inputs)
